# baseline (speedup 1.0000x reference)
import functools

import jax
import jax.numpy as jnp
from jax import lax
from jax.experimental import pallas as pl
from jax.experimental.pallas import tpu as pltpu
from jax.experimental.pallas import tpu_sc as plsc

NC, NS, LANES = 2, 16, 16
NW = NC * NS

B, N, L, D = 4096, 200, 50, 128
BATCH_PER_W = B // NW
PER_W = L * BATCH_PER_W
CHUNK = 128
NBUF = 7


def _make_mesh():
    return plsc.VectorSubcoreMesh(
        core_axis_name="c", subcore_axis_name="s",
        num_cores=NC, num_subcores=NS)


@functools.partial(
    pl.kernel,
    out_type=jax.ShapeDtypeStruct((L * B, D), jnp.float32),
    mesh=_make_mesh(),
    compiler_params=pltpu.CompilerParams(use_tc_tiling_on_sc=True),
    scratch_types=[
        pltpu.VMEM((L, CHUNK), jnp.int32),
        pltpu.VMEM((NBUF, CHUNK, D), jnp.float32),
        pltpu.SemaphoreType.DMA((NBUF,)),
        pltpu.SemaphoreType.DMA((NBUF,)),
    ],
)
def _sc_gather(data_hbm, idx_hbm, out_hbm, idx_v, rows_v, gsem, osem):
    w = lax.axis_index("s") * NC + lax.axis_index("c")
    batch_base = w * BATCH_PER_W

    pltpu.sync_copy(idx_hbm.at[w], idx_v)

    lane = lax.iota(jnp.int32, LANES)

    def fix_chunk(l):
        for v in range(CHUNK // LANES):
            b = batch_base + v * LANES + lane
            sl = pl.ds(v * LANES, LANES)
            idx_v[l, sl] = idx_v[l, sl] + b * N

    def start_gather(l, buf):
        pltpu.async_copy(
            data_hbm.at[idx_v.at[l]],
            rows_v.at[buf], gsem.at[buf])

    def wait_gather(buf):
        pltpu.make_async_copy(
            data_hbm.at[pl.ds(0, CHUNK)], rows_v.at[buf], gsem.at[buf]).wait()

    def start_put(l, buf):
        out_off = pl.multiple_of(l * B + batch_base, CHUNK)
        pltpu.async_copy(
            rows_v.at[buf], out_hbm.at[pl.ds(out_off, CHUNK)], osem.at[buf])

    def wait_put(buf):
        pltpu.make_async_copy(
            rows_v.at[buf], out_hbm.at[pl.ds(0, CHUNK)], osem.at[buf]).wait()

    for g in range(NBUF):
        fix_chunk(g)
        start_gather(g, g)

    def step(l, buf):
        wait_gather(buf)
        start_put(l, buf)

        @pl.when(l + NBUF < L)
        def _():
            wait_put(buf)
            fix_chunk(l + NBUF)
            start_gather(l + NBUF, buf)

        nxt = buf + 1
        return lax.select(nxt == NBUF, 0, nxt)

    lax.fori_loop(0, L, step, 0)

    for b in range(NBUF):
        wait_put(b)


def kernel(data, indices):
    data_flat = data.reshape(B * N, D)
    idx_blocks = (indices.astype(jnp.int32).T
                  .reshape(L, NW, BATCH_PER_W)
                  .transpose(1, 0, 2)
                  .reshape(NW, L, CHUNK))
    out_flat = _sc_gather(data_flat, idx_blocks)
    return out_flat.reshape(L, B, D).transpose(1, 0, 2)


# speedup vs baseline: 1.0237x; 1.0237x over previous
import functools

import jax
import jax.numpy as jnp
from jax import lax
from jax.experimental import pallas as pl
from jax.experimental.pallas import tpu as pltpu
from jax.experimental.pallas import tpu_sc as plsc

NC, NS, LANES = 2, 16, 16
NW = NC * NS

B, N, L, D = 4096, 200, 50, 128
BATCH_PER_W = B // NW
PER_W = L * BATCH_PER_W
CHUNK = 128
NBUF = 6


def _make_mesh():
    return plsc.VectorSubcoreMesh(
        core_axis_name="c", subcore_axis_name="s",
        num_cores=NC, num_subcores=NS)


@functools.partial(
    pl.kernel,
    out_type=jax.ShapeDtypeStruct((L * B, D), jnp.float32),
    mesh=_make_mesh(),
    compiler_params=pltpu.CompilerParams(use_tc_tiling_on_sc=True),
    scratch_types=[
        pltpu.VMEM((L, CHUNK), jnp.int32),
        pltpu.VMEM((NBUF, CHUNK, D), jnp.float32),
        pltpu.SemaphoreType.DMA((NBUF,)),
        pltpu.SemaphoreType.DMA((NBUF,)),
    ],
)
def _sc_gather(data_hbm, idx_hbm, out_hbm, idx_v, rows_v, gsem, osem):
    w = lax.axis_index("s") * NC + lax.axis_index("c")
    batch_base = w * BATCH_PER_W

    pltpu.sync_copy(idx_hbm.at[w], idx_v)

    lane = lax.iota(jnp.int32, LANES)

    def fix_chunk(l):
        for v in range(CHUNK // LANES):
            b = batch_base + v * LANES + lane
            sl = pl.ds(v * LANES, LANES)
            idx_v[l, sl] = idx_v[l, sl] + b * N

    def start_gather(l, buf):
        pltpu.async_copy(
            data_hbm.at[idx_v.at[l]],
            rows_v.at[buf], gsem.at[buf])

    def wait_gather(buf):
        pltpu.make_async_copy(
            data_hbm.at[pl.ds(0, CHUNK)], rows_v.at[buf], gsem.at[buf]).wait()

    def start_put(l, buf):
        out_off = pl.multiple_of(l * B + batch_base, CHUNK)
        pltpu.async_copy(
            rows_v.at[buf], out_hbm.at[pl.ds(out_off, CHUNK)], osem.at[buf])

    def wait_put(buf):
        pltpu.make_async_copy(
            rows_v.at[buf], out_hbm.at[pl.ds(0, CHUNK)], osem.at[buf]).wait()

    for g in range(NBUF):
        fix_chunk(g)
        start_gather(g, g)

    def step(l, buf):
        wait_gather(buf)
        start_put(l, buf)

        @pl.when(l + NBUF < L)
        def _():
            wait_put(buf)
            fix_chunk(l + NBUF)
            start_gather(l + NBUF, buf)

        nxt = buf + 1
        return lax.select(nxt == NBUF, 0, nxt)

    lax.fori_loop(0, L, step, 0)

    for b in range(NBUF):
        wait_put(b)


def kernel(data, indices):
    data_flat = data.reshape(B * N, D)
    idx_blocks = (indices.astype(jnp.int32).T
                  .reshape(L, NW, BATCH_PER_W)
                  .transpose(1, 0, 2)
                  .reshape(NW, L, CHUNK))
    out_flat = _sc_gather(data_flat, idx_blocks)
    return out_flat.reshape(L, B, D).transpose(1, 0, 2)
